# baseline (speedup 1.0000x reference)
import jax
import jax.numpy as jnp
from jax.experimental import pallas as pl
from jax.experimental.pallas import tpu as pltpu

B, NCOILS = 16, 8
H = W = 256
MH = MW = 64
KH = KW = 3
NSTACK = 2
K_PER = 512
NCH = 2 * NCOILS
NTAP = KH * KW
KDIM = NCH * NTAP
HCHUNK = 64
HI_BITS = 0x3F800001


def _net_kernel(x2_ref, wmat_ref, xr_ref, xi_ref, out_ref, xp_ref, xq_ref):
    x2 = x2_ref[0]
    bits = jax.lax.bitcast_convert_type(x2, jnp.int32)

    def bisect(_, lohi):
        lo, hi = lohi
        mid = jax.lax.div(lo + hi, jnp.int32(2))
        cnt = jnp.sum((bits >= mid).astype(jnp.int32))
        big = cnt >= K_PER
        return (jnp.where(big, mid, lo), jnp.where(big, hi, mid))

    lo, _ = jax.lax.fori_loop(0, 31, bisect,
                              (jnp.int32(0), jnp.int32(HI_BITS)))
    mask64 = jnp.logical_and(bits >= lo, x2 > 0.0).astype(jnp.float32)
    mask = jnp.tile(mask64, (H // MH, W // MW))

    xr = xr_ref[0]
    xi = xi_ref[0]
    m3 = mask[None, :, :]

    for ref in (xp_ref, xq_ref):
        ref[:, 0:1, :] = jnp.zeros((NCH, 1, W + 2), jnp.float32)
        ref[:, H + 1:H + 2, :] = jnp.zeros((NCH, 1, W + 2), jnp.float32)
        ref[:, :, 0:1] = jnp.zeros((NCH, H + 2, 1), jnp.float32)
        ref[:, :, W + 1:W + 2] = jnp.zeros((NCH, H + 2, 1), jnp.float32)

    xp_ref[:NCOILS, 1:H + 1, 1:W + 1] = xr * m3
    xp_ref[NCOILS:, 1:H + 1, 1:W + 1] = xi * m3

    for s in range(NSTACK):
        src = xp_ref if s == 0 else xq_ref
        wmat = wmat_ref[s]
        for h0 in range(0, H, HCHUNK):
            parts = []
            for dy in range(KH):
                for dx in range(KW):
                    sl = src[:, h0 + dy:h0 + dy + HCHUNK, dx:dx + W]
                    parts.append(sl.reshape(NCH, HCHUNK * W))
            a = jnp.concatenate(parts, axis=0)
            o = jax.lax.dot_general(
                wmat, a, (((1,), (0,)), ((), ())),
                preferred_element_type=jnp.float32)
            o = o.reshape(NCH, HCHUNK, W)
            mch = m3[:, h0:h0 + HCHUNK, :]
            xch = jnp.concatenate(
                [xr[:, h0:h0 + HCHUNK, :], xi[:, h0:h0 + HCHUNK, :]], axis=0)
            pred = jnp.where(mch > 0.5, xch, o)
            if s == 0:
                xq_ref[:, 1 + h0:1 + h0 + HCHUNK, 1:W + 1] = pred
            else:
                out_ref[0, :, h0:h0 + HCHUNK, :] = pred


@jax.jit
def kernel(x, my_input_1, conv_mask_W, Wr, Wi):
    xr = x[..., 0]
    xi = x[..., 1]
    conv_out = jax.lax.conv_transpose(
        my_input_1, conv_mask_W, strides=(1, 1), padding='VALID',
        dimension_numbers=('NCHW', 'IOHW', 'NCHW'), transpose_kernel=True)
    x2 = jax.nn.sigmoid(conv_out).reshape(B, MH, MW)

    wr0 = Wr.at[..., KH // 2, KW // 2].set(0.0)
    wi0 = Wi.at[..., KH // 2, KW // 2].set(0.0)
    top = jnp.concatenate([wr0, -wi0], axis=2)
    bot = jnp.concatenate([wi0, wr0], axis=2)
    mfull = jnp.concatenate([top, bot], axis=1)
    wmat = mfull.transpose(0, 1, 3, 4, 2).reshape(NSTACK, NCH, KDIM)

    out16 = pl.pallas_call(
        _net_kernel,
        grid=(B,),
        in_specs=[
            pl.BlockSpec((1, MH, MW), lambda b: (b, 0, 0)),
            pl.BlockSpec((NSTACK, NCH, KDIM), lambda b: (0, 0, 0)),
            pl.BlockSpec((1, NCOILS, H, W), lambda b: (b, 0, 0, 0)),
            pl.BlockSpec((1, NCOILS, H, W), lambda b: (b, 0, 0, 0)),
        ],
        out_specs=pl.BlockSpec((1, NCH, H, W), lambda b: (b, 0, 0, 0)),
        out_shape=jax.ShapeDtypeStruct((B, NCH, H, W), jnp.float32),
        scratch_shapes=[
            pltpu.VMEM((NCH, H + 2, W + 2), jnp.float32),
            pltpu.VMEM((NCH, H + 2, W + 2), jnp.float32),
        ],
        compiler_params=pltpu.CompilerParams(
            dimension_semantics=("parallel",)),
    )(x2, wmat, xr, xi)

    return jnp.stack((out16[:, :NCOILS], out16[:, NCOILS:]), axis=-1)


# speedup vs baseline: 1.1214x; 1.1214x over previous
import jax
import jax.numpy as jnp
from jax.experimental import pallas as pl
from jax.experimental.pallas import tpu as pltpu

B, NCOILS = 16, 8
H = W = 256
MH = MW = 64
KH = KW = 3
NSTACK = 2
K_PER = 512
NCH = 2 * NCOILS
NTAP = KH * KW
KDIM = NCH * NTAP
HCHUNK = 64
HI_BITS = 0x3F800001


def _net_kernel(x2_ref, wmat_ref, xr_ref, xi_ref, out_ref, xp_ref, xq_ref):
    x2 = x2_ref[0]
    bits = jax.lax.bitcast_convert_type(x2, jnp.int32)

    def bisect(_, lohi):
        lo, hi = lohi
        mid = jax.lax.div(lo + hi, jnp.int32(2))
        cnt = jnp.sum((bits >= mid).astype(jnp.int32))
        big = cnt >= K_PER
        return (jnp.where(big, mid, lo), jnp.where(big, hi, mid))

    lo, _ = jax.lax.fori_loop(0, 31, bisect,
                              (jnp.int32(0), jnp.int32(HI_BITS)))
    mask64 = jnp.logical_and(bits >= lo, x2 > 0.0).astype(jnp.float32)
    mask = jnp.tile(mask64, (H // MH, W // MW))

    xr = xr_ref[0]
    xi = xi_ref[0]
    m3 = mask[None, :, :]

    for ref in (xp_ref, xq_ref):
        ref[:, 0:1, :] = jnp.zeros((NCH, 1, W + 2), jnp.bfloat16)
        ref[:, H + 1:H + 2, :] = jnp.zeros((NCH, 1, W + 2), jnp.bfloat16)
        ref[:, :, 0:1] = jnp.zeros((NCH, H + 2, 1), jnp.bfloat16)
        ref[:, :, W + 1:W + 2] = jnp.zeros((NCH, H + 2, 1), jnp.bfloat16)

    xp_ref[:NCOILS, 1:H + 1, 1:W + 1] = (xr * m3).astype(jnp.bfloat16)
    xp_ref[NCOILS:, 1:H + 1, 1:W + 1] = (xi * m3).astype(jnp.bfloat16)

    for s in range(NSTACK):
        src = xp_ref if s == 0 else xq_ref
        wmat = wmat_ref[s]
        for h0 in range(0, H, HCHUNK):
            parts = []
            for dy in range(KH):
                for dx in range(KW):
                    sl = src[:, h0 + dy:h0 + dy + HCHUNK, dx:dx + W]
                    parts.append(sl.reshape(NCH, HCHUNK * W))
            a = jnp.concatenate(parts, axis=0)
            o = jax.lax.dot_general(
                wmat, a, (((1,), (0,)), ((), ())),
                preferred_element_type=jnp.float32)
            o = o.reshape(NCH, HCHUNK, W)
            mch = m3[:, h0:h0 + HCHUNK, :]
            xch = jnp.concatenate(
                [xr[:, h0:h0 + HCHUNK, :], xi[:, h0:h0 + HCHUNK, :]], axis=0)
            if s == 0:
                pred = jnp.where(mch > 0.5, xch, o)
                xq_ref[:, 1 + h0:1 + h0 + HCHUNK, 1:W + 1] = (
                    pred.astype(jnp.bfloat16))
            else:
                out_ref[0, :, h0:h0 + HCHUNK, :] = jnp.where(mch > 0.5, xch, o)


@jax.jit
def kernel(x, my_input_1, conv_mask_W, Wr, Wi):
    xr = x[..., 0]
    xi = x[..., 1]
    conv_out = jax.lax.conv_transpose(
        my_input_1, conv_mask_W, strides=(1, 1), padding='VALID',
        dimension_numbers=('NCHW', 'IOHW', 'NCHW'), transpose_kernel=True)
    x2 = jax.nn.sigmoid(conv_out).reshape(B, MH, MW)

    wr0 = Wr.at[..., KH // 2, KW // 2].set(0.0)
    wi0 = Wi.at[..., KH // 2, KW // 2].set(0.0)
    top = jnp.concatenate([wr0, -wi0], axis=2)
    bot = jnp.concatenate([wi0, wr0], axis=2)
    mfull = jnp.concatenate([top, bot], axis=1)
    wmat = mfull.transpose(0, 1, 3, 4, 2).reshape(NSTACK, NCH, KDIM)
    wmat = wmat.astype(jnp.bfloat16)

    out16 = pl.pallas_call(
        _net_kernel,
        grid=(B,),
        in_specs=[
            pl.BlockSpec((1, MH, MW), lambda b: (b, 0, 0)),
            pl.BlockSpec((NSTACK, NCH, KDIM), lambda b: (0, 0, 0)),
            pl.BlockSpec((1, NCOILS, H, W), lambda b: (b, 0, 0, 0)),
            pl.BlockSpec((1, NCOILS, H, W), lambda b: (b, 0, 0, 0)),
        ],
        out_specs=pl.BlockSpec((1, NCH, H, W), lambda b: (b, 0, 0, 0)),
        out_shape=jax.ShapeDtypeStruct((B, NCH, H, W), jnp.float32),
        scratch_shapes=[
            pltpu.VMEM((NCH, H + 2, W + 2), jnp.bfloat16),
            pltpu.VMEM((NCH, H + 2, W + 2), jnp.bfloat16),
        ],
        compiler_params=pltpu.CompilerParams(
            dimension_semantics=("parallel",)),
    )(x2, wmat, xr, xi)

    return jnp.stack((out16[:, :NCOILS], out16[:, NCOILS:]), axis=-1)


# speedup vs baseline: 1.5436x; 1.3765x over previous
import jax
import jax.numpy as jnp
from jax.experimental import pallas as pl
from jax.experimental.pallas import tpu as pltpu

B, NCOILS = 16, 8
H = W = 256
MH = MW = 64
KH = KW = 3
NSTACK = 2
K_PER = 512
NCH = 2 * NCOILS
NTAP = KH * KW
KDIM = NCH * NTAP
HW = H * W
PAD = 2 * W
FW = HW + 2 * PAD
NCHUNK = 4
CW = HW // NCHUNK
HCH = H // NCHUNK
HI_BITS = 0x3F800001


def _net_kernel(x2_ref, wmat_ref, xr_ref, xi_ref, out_ref,
                xa_ref, xb_ref, xl_ref, xr2_ref):
    x2 = x2_ref[0]
    bits = jax.lax.bitcast_convert_type(x2, jnp.int32)

    def bisect(_, lohi):
        lo, hi = lohi
        mid = jax.lax.div(lo + hi, jnp.int32(2))
        cnt = jnp.sum((bits >= mid).astype(jnp.int32))
        big = cnt >= K_PER
        return (jnp.where(big, mid, lo), jnp.where(big, hi, mid))

    lo, _ = jax.lax.fori_loop(0, 31, bisect,
                              (jnp.int32(0), jnp.int32(HI_BITS)))
    mask64 = jnp.logical_and(bits >= lo, x2 > 0.0).astype(jnp.float32)
    mask = jnp.tile(mask64, (H // MH, W // MW))

    xr = xr_ref[0]
    xi = xi_ref[0]
    m3 = mask[None, :, :]

    @pl.when(pl.program_id(0) == 0)
    def _():
        for ref in (xa_ref, xb_ref, xl_ref, xr2_ref):
            ref[:, :PAD] = jnp.zeros((NCH, PAD), jnp.bfloat16)
            ref[:, PAD + HW:] = jnp.zeros((NCH, PAD), jnp.bfloat16)

    mr = (xr * m3).astype(jnp.bfloat16).reshape(NCOILS, HW)
    mi = (xi * m3).astype(jnp.bfloat16).reshape(NCOILS, HW)
    xa_ref[:NCOILS, PAD:PAD + HW] = mr
    xa_ref[NCOILS:, PAD:PAD + HW] = mi

    lidx = jax.lax.broadcasted_iota(jnp.int32, (1, HW), 1)
    wpos = jax.lax.rem(lidx, jnp.int32(W))
    mleft = (wpos != 0).astype(jnp.bfloat16)
    mright = (wpos != W - 1).astype(jnp.bfloat16)
    mflat = mask.reshape(1, HW) > 0.5

    for s in range(NSTACK):
        src = xa_ref if s == 0 else xb_ref
        x0 = src[:, PAD:PAD + HW]
        xl_ref[:, PAD:PAD + HW] = x0 * mleft
        xr2_ref[:, PAD:PAD + HW] = x0 * mright
        wmat = wmat_ref[s]
        for ci in range(NCHUNK):
            c0 = PAD + ci * CW
            parts = []
            for dy in range(KH):
                base = c0 + W * (dy - 1)
                parts.append(xr2_ref[:, base - 1:base - 1 + CW])
                parts.append(src[:, base:base + CW])
                parts.append(xl_ref[:, base + 1:base + 1 + CW])
            a = jnp.concatenate(parts, axis=0)
            o = jax.lax.dot_general(
                wmat, a, (((1,), (0,)), ((), ())),
                preferred_element_type=jnp.float32)
            mch = mflat[:, ci * CW:(ci + 1) * CW]
            if s == 0:
                x0ch = x0[:, ci * CW:(ci + 1) * CW]
                pred = jnp.where(mch, x0ch.astype(jnp.float32), o)
                xb_ref[:, c0:c0 + CW] = pred.astype(jnp.bfloat16)
            else:
                h0 = ci * HCH
                ohw = o.reshape(NCH, HCH, W)
                mhw = m3[:, h0:h0 + HCH, :]
                xch = jnp.concatenate(
                    [xr[:, h0:h0 + HCH, :], xi[:, h0:h0 + HCH, :]], axis=0)
                out_ref[0, :, h0:h0 + HCH, :] = jnp.where(mhw > 0.5, xch, ohw)


@jax.jit
def kernel(x, my_input_1, conv_mask_W, Wr, Wi):
    xr = x[..., 0]
    xi = x[..., 1]
    conv_out = jax.lax.conv_transpose(
        my_input_1, conv_mask_W, strides=(1, 1), padding='VALID',
        dimension_numbers=('NCHW', 'IOHW', 'NCHW'), transpose_kernel=True)
    x2 = jax.nn.sigmoid(conv_out).reshape(B, MH, MW)

    wr0 = Wr.at[..., KH // 2, KW // 2].set(0.0)
    wi0 = Wi.at[..., KH // 2, KW // 2].set(0.0)
    top = jnp.concatenate([wr0, -wi0], axis=2)
    bot = jnp.concatenate([wi0, wr0], axis=2)
    mfull = jnp.concatenate([top, bot], axis=1)
    wmat = mfull.transpose(0, 1, 3, 4, 2).reshape(NSTACK, NCH, KDIM)
    wmat = wmat.astype(jnp.bfloat16)

    out16 = pl.pallas_call(
        _net_kernel,
        grid=(B,),
        in_specs=[
            pl.BlockSpec((1, MH, MW), lambda b: (b, 0, 0)),
            pl.BlockSpec((NSTACK, NCH, KDIM), lambda b: (0, 0, 0)),
            pl.BlockSpec((1, NCOILS, H, W), lambda b: (b, 0, 0, 0)),
            pl.BlockSpec((1, NCOILS, H, W), lambda b: (b, 0, 0, 0)),
        ],
        out_specs=pl.BlockSpec((1, NCH, H, W), lambda b: (b, 0, 0, 0)),
        out_shape=jax.ShapeDtypeStruct((B, NCH, H, W), jnp.float32),
        scratch_shapes=[
            pltpu.VMEM((NCH, FW), jnp.bfloat16),
            pltpu.VMEM((NCH, FW), jnp.bfloat16),
            pltpu.VMEM((NCH, FW), jnp.bfloat16),
            pltpu.VMEM((NCH, FW), jnp.bfloat16),
        ],
    )(x2, wmat, xr, xi)

    return jnp.stack((out16[:, :NCOILS], out16[:, NCOILS:]), axis=-1)


# speedup vs baseline: 1.7721x; 1.1480x over previous
import jax
import jax.numpy as jnp
from jax.experimental import pallas as pl
from jax.experimental.pallas import tpu as pltpu

B, NCOILS = 16, 8
H = W = 256
MH = MW = 64
KH = KW = 3
NSTACK = 2
K_PER = 512
NCH = 2 * NCOILS
NTAP = KH * KW
KDIM = NCH * NTAP
HW = H * W
PAD = 2 * W
FW = HW + 2 * PAD
NCHUNK = 4
CW = HW // NCHUNK
HCH = H // NCHUNK
HI_BITS = 0x3F800001


def _net_kernel(x2_ref, wmat_ref, xr_ref, xi_ref, out_ref, mask_ref,
                xa_ref, xb_ref, xl_ref, xr2_ref):
    x2 = x2_ref[0]
    bits = jax.lax.bitcast_convert_type(x2, jnp.int32)

    def bisect(_, lohi):
        lo, hi = lohi
        mid = jax.lax.div(lo + hi, jnp.int32(2))
        cnt = jnp.sum((bits >= mid).astype(jnp.int32))
        big = cnt >= K_PER
        return (jnp.where(big, mid, lo), jnp.where(big, hi, mid))

    lo, _ = jax.lax.fori_loop(0, 31, bisect,
                              (jnp.int32(0), jnp.int32(HI_BITS)))
    mask64 = jnp.logical_and(bits >= lo, x2 > 0.0).astype(jnp.float32)
    mask = jnp.tile(mask64, (H // MH, W // MW))
    mask_ref[0] = mask

    xr = xr_ref[0]
    xi = xi_ref[0]
    m3 = mask.astype(jnp.bfloat16)[None, :, :]

    @pl.when(pl.program_id(0) == 0)
    def _():
        for ref in (xa_ref, xb_ref, xl_ref, xr2_ref):
            ref[:, :PAD] = jnp.zeros((NCH, PAD), jnp.bfloat16)
            ref[:, PAD + HW:] = jnp.zeros((NCH, PAD), jnp.bfloat16)

    mr = (xr * m3).reshape(NCOILS, HW)
    mi = (xi * m3).reshape(NCOILS, HW)
    xa_ref[:NCOILS, PAD:PAD + HW] = mr
    xa_ref[NCOILS:, PAD:PAD + HW] = mi

    lidx = jax.lax.broadcasted_iota(jnp.int32, (1, HW), 1)
    wpos = jax.lax.rem(lidx, jnp.int32(W))
    mleft = (wpos != 0).astype(jnp.bfloat16)
    mright = (wpos != W - 1).astype(jnp.bfloat16)
    mflat = mask.reshape(1, HW) > 0.5

    for s in range(NSTACK):
        src = xa_ref if s == 0 else xb_ref
        x0 = src[:, PAD:PAD + HW]
        xl_ref[:, PAD:PAD + HW] = x0 * mleft
        xr2_ref[:, PAD:PAD + HW] = x0 * mright
        wmat = wmat_ref[s]
        for ci in range(NCHUNK):
            c0 = PAD + ci * CW
            parts = []
            for dy in range(KH):
                base = c0 + W * (dy - 1)
                parts.append(xr2_ref[:, base - 1:base - 1 + CW])
                parts.append(src[:, base:base + CW])
                parts.append(xl_ref[:, base + 1:base + 1 + CW])
            a = jnp.concatenate(parts, axis=0)
            o = jax.lax.dot_general(
                wmat, a, (((1,), (0,)), ((), ())),
                preferred_element_type=jnp.float32)
            if s == 0:
                mch = mflat[:, ci * CW:(ci + 1) * CW]
                x0ch = x0[:, ci * CW:(ci + 1) * CW]
                pred = jnp.where(mch, x0ch, o.astype(jnp.bfloat16))
                xb_ref[:, c0:c0 + CW] = pred
            else:
                h0 = ci * HCH
                out_ref[0, :, h0:h0 + HCH, :] = (
                    o.astype(jnp.bfloat16).reshape(NCH, HCH, W))


@jax.jit
def kernel(x, my_input_1, conv_mask_W, Wr, Wi):
    xr = x[..., 0].astype(jnp.bfloat16)
    xi = x[..., 1].astype(jnp.bfloat16)
    conv_out = jax.lax.conv_transpose(
        my_input_1, conv_mask_W, strides=(1, 1), padding='VALID',
        dimension_numbers=('NCHW', 'IOHW', 'NCHW'), transpose_kernel=True)
    x2 = jax.nn.sigmoid(conv_out).reshape(B, MH, MW)

    wr0 = Wr.at[..., KH // 2, KW // 2].set(0.0)
    wi0 = Wi.at[..., KH // 2, KW // 2].set(0.0)
    top = jnp.concatenate([wr0, -wi0], axis=2)
    bot = jnp.concatenate([wi0, wr0], axis=2)
    mfull = jnp.concatenate([top, bot], axis=1)
    wmat = mfull.transpose(0, 1, 3, 4, 2).reshape(NSTACK, NCH, KDIM)
    wmat = wmat.astype(jnp.bfloat16)

    out16, maskb = pl.pallas_call(
        _net_kernel,
        grid=(B,),
        in_specs=[
            pl.BlockSpec((1, MH, MW), lambda b: (b, 0, 0)),
            pl.BlockSpec((NSTACK, NCH, KDIM), lambda b: (0, 0, 0)),
            pl.BlockSpec((1, NCOILS, H, W), lambda b: (b, 0, 0, 0)),
            pl.BlockSpec((1, NCOILS, H, W), lambda b: (b, 0, 0, 0)),
        ],
        out_specs=[
            pl.BlockSpec((1, NCH, H, W), lambda b: (b, 0, 0, 0)),
            pl.BlockSpec((1, H, W), lambda b: (b, 0, 0)),
        ],
        out_shape=[
            jax.ShapeDtypeStruct((B, NCH, H, W), jnp.bfloat16),
            jax.ShapeDtypeStruct((B, H, W), jnp.float32),
        ],
        scratch_shapes=[
            pltpu.VMEM((NCH, FW), jnp.bfloat16),
            pltpu.VMEM((NCH, FW), jnp.bfloat16),
            pltpu.VMEM((NCH, FW), jnp.bfloat16),
            pltpu.VMEM((NCH, FW), jnp.bfloat16),
        ],
    )(x2, wmat, xr, xi)

    conv5 = jnp.stack((out16[:, :NCOILS], out16[:, NCOILS:]),
                      axis=-1).astype(jnp.float32)
    sel = (maskb > 0.5)[:, None, :, :, None]
    return jnp.where(sel, x, conv5)


# speedup vs baseline: 2.1114x; 1.1915x over previous
import functools

import jax
import jax.numpy as jnp
from jax import lax
from jax.experimental import pallas as pl
from jax.experimental.pallas import tpu as pltpu
from jax.experimental.pallas import tpu_sc as plsc

B, NCOILS = 16, 8
H = W = 256
MH = MW = 64
KH = KW = 3
NSTACK = 2
K_PER = 512
NCH = 2 * NCOILS
NTAP = KH * KW
KDIM = NCH * NTAP
HW = H * W
PAD = 2 * W
FW = HW + 2 * PAD
NCHUNK = 4
CW = HW // NCHUNK
HCH = H // NCHUNK
HI_BITS = 0x3F800001
NC, NS, LANES = 2, 16, 16
NROW = MH * MW


@functools.partial(
    pl.kernel,
    mesh=plsc.VectorSubcoreMesh(core_axis_name="c", subcore_axis_name="s"),
    out_type=jax.ShapeDtypeStruct((B, LANES), jnp.int32),
    scratch_types=[
        pltpu.VMEM((NROW,), jnp.int32),
        pltpu.VMEM((LANES,), jnp.int32),
    ],
    compiler_params=pltpu.CompilerParams(needs_layout_passes=False),
)
def _topk_thresh_sc(x2i_hbm, out_hbm, xrow_v, tmp_v):
    wid = lax.axis_index("s") * NC + lax.axis_index("c")

    @pl.when(wid < B)
    def _():
        pltpu.sync_copy(x2i_hbm.at[wid], xrow_v)
        idx15 = jnp.full((LANES,), LANES - 1, jnp.int32)

        def bisect(_, lohi):
            lo, hi = lohi
            mid = lax.shift_right_logical(lo + hi, 1)

            def count(j, acc):
                v = xrow_v[pl.ds(j * LANES, LANES)]
                return acc + jnp.where(v >= mid, 1, 0).astype(jnp.int32)

            acc = lax.fori_loop(0, NROW // LANES, count,
                                jnp.zeros((LANES,), jnp.int32))
            tmp_v[...] = plsc.cumsum(acc)
            tot = plsc.load_gather(tmp_v, [idx15])
            big = tot >= K_PER
            return (jnp.where(big, mid, lo), jnp.where(big, hi, mid))

        lo, _ = lax.fori_loop(
            0, 31, bisect,
            (jnp.zeros((LANES,), jnp.int32),
             jnp.full((LANES,), HI_BITS, jnp.int32)))
        tmp_v[...] = lo
        pltpu.sync_copy(tmp_v, out_hbm.at[wid])


def _net_kernel(x2_ref, thr_ref, wmat_ref, xr_ref, xi_ref, out_ref, mask_ref,
                xa_ref, xb_ref, xl_ref, xr2_ref):
    x2 = x2_ref[0]
    bits = jax.lax.bitcast_convert_type(x2, jnp.int32)
    lo = thr_ref[0, 0, 0]
    mask64 = jnp.logical_and(bits >= lo, x2 > 0.0).astype(jnp.float32)
    mask = jnp.tile(mask64, (H // MH, W // MW))
    mask_ref[0] = mask

    xr = xr_ref[0]
    xi = xi_ref[0]
    m3 = mask.astype(jnp.bfloat16)[None, :, :]

    @pl.when(pl.program_id(0) == 0)
    def _():
        for ref in (xa_ref, xb_ref, xl_ref, xr2_ref):
            ref[:, :PAD] = jnp.zeros((NCH, PAD), jnp.bfloat16)
            ref[:, PAD + HW:] = jnp.zeros((NCH, PAD), jnp.bfloat16)

    mr = (xr * m3).reshape(NCOILS, HW)
    mi = (xi * m3).reshape(NCOILS, HW)
    xa_ref[:NCOILS, PAD:PAD + HW] = mr
    xa_ref[NCOILS:, PAD:PAD + HW] = mi

    lidx = jax.lax.broadcasted_iota(jnp.int32, (1, HW), 1)
    wpos = jax.lax.rem(lidx, jnp.int32(W))
    mleft = (wpos != 0).astype(jnp.bfloat16)
    mright = (wpos != W - 1).astype(jnp.bfloat16)
    mflat = mask.reshape(1, HW) > 0.5

    for s in range(NSTACK):
        src = xa_ref if s == 0 else xb_ref
        x0 = src[:, PAD:PAD + HW]
        xl_ref[:, PAD:PAD + HW] = x0 * mleft
        xr2_ref[:, PAD:PAD + HW] = x0 * mright
        wmat = wmat_ref[s]
        for ci in range(NCHUNK):
            c0 = PAD + ci * CW
            parts = []
            for dy in range(KH):
                base = c0 + W * (dy - 1)
                parts.append(xr2_ref[:, base - 1:base - 1 + CW])
                parts.append(src[:, base:base + CW])
                parts.append(xl_ref[:, base + 1:base + 1 + CW])
            a = jnp.concatenate(parts, axis=0)
            o = jax.lax.dot_general(
                wmat, a, (((1,), (0,)), ((), ())),
                preferred_element_type=jnp.float32)
            if s == 0:
                mch = mflat[:, ci * CW:(ci + 1) * CW]
                x0ch = x0[:, ci * CW:(ci + 1) * CW]
                pred = jnp.where(mch, x0ch, o.astype(jnp.bfloat16))
                xb_ref[:, c0:c0 + CW] = pred
            else:
                h0 = ci * HCH
                out_ref[0, :, h0:h0 + HCH, :] = (
                    o.astype(jnp.bfloat16).reshape(NCH, HCH, W))


@jax.jit
def kernel(x, my_input_1, conv_mask_W, Wr, Wi):
    xr = x[..., 0].astype(jnp.bfloat16)
    xi = x[..., 1].astype(jnp.bfloat16)
    conv_out = jax.lax.conv_transpose(
        my_input_1, conv_mask_W, strides=(1, 1), padding='VALID',
        dimension_numbers=('NCHW', 'IOHW', 'NCHW'), transpose_kernel=True)
    x2 = jax.nn.sigmoid(conv_out).reshape(B, MH, MW)
    x2i = jax.lax.bitcast_convert_type(x2.reshape(B, NROW), jnp.int32)
    thr = _topk_thresh_sc(x2i).reshape(B, 1, LANES)

    wr0 = Wr.at[..., KH // 2, KW // 2].set(0.0)
    wi0 = Wi.at[..., KH // 2, KW // 2].set(0.0)
    top = jnp.concatenate([wr0, -wi0], axis=2)
    bot = jnp.concatenate([wi0, wr0], axis=2)
    mfull = jnp.concatenate([top, bot], axis=1)
    wmat = mfull.transpose(0, 1, 3, 4, 2).reshape(NSTACK, NCH, KDIM)
    wmat = wmat.astype(jnp.bfloat16)

    out16, maskb = pl.pallas_call(
        _net_kernel,
        grid=(B,),
        in_specs=[
            pl.BlockSpec((1, MH, MW), lambda b: (b, 0, 0)),
            pl.BlockSpec((1, 1, LANES), lambda b: (b, 0, 0)),
            pl.BlockSpec((NSTACK, NCH, KDIM), lambda b: (0, 0, 0)),
            pl.BlockSpec((1, NCOILS, H, W), lambda b: (b, 0, 0, 0)),
            pl.BlockSpec((1, NCOILS, H, W), lambda b: (b, 0, 0, 0)),
        ],
        out_specs=[
            pl.BlockSpec((1, NCH, H, W), lambda b: (b, 0, 0, 0)),
            pl.BlockSpec((1, H, W), lambda b: (b, 0, 0)),
        ],
        out_shape=[
            jax.ShapeDtypeStruct((B, NCH, H, W), jnp.bfloat16),
            jax.ShapeDtypeStruct((B, H, W), jnp.float32),
        ],
        scratch_shapes=[
            pltpu.VMEM((NCH, FW), jnp.bfloat16),
            pltpu.VMEM((NCH, FW), jnp.bfloat16),
            pltpu.VMEM((NCH, FW), jnp.bfloat16),
            pltpu.VMEM((NCH, FW), jnp.bfloat16),
        ],
    )(x2, thr, wmat, xr, xi)

    conv5 = jnp.stack((out16[:, :NCOILS], out16[:, NCOILS:]),
                      axis=-1).astype(jnp.float32)
    sel = (maskb > 0.5)[:, None, :, :, None]
    return jnp.where(sel, x, conv5)
